# baseline (speedup 1.0000x reference)
import functools

import jax
import jax.numpy as jnp
from jax import lax
from jax.experimental import pallas as pl
from jax.experimental.pallas import tpu as pltpu
from jax.experimental.pallas import tpu_sc as plsc

N = 10000
E = 320000
D = 128
NPAD = 10240
NC = 2
NS = 16
EDGES_PER_SC = E // NC
EDGES_PER_TILE = EDGES_PER_SC // NS
BATCH = 80
NBATCH = EDGES_PER_TILE // BATCH
ROWS_PER_TILE = NPAD // NS
FLUSH_ROWS = 128
PRESERVE_C = 0.1

_MESH = dict(core_axis_name="c", subcore_axis_name="s")


def _zero_vec_ref(ref, nwords):
    zeros = jnp.zeros((16,), jnp.float32)

    def body(i, _):
        ref[pl.ds(i * 16, 16)] = zeros
        return 0

    lax.fori_loop(0, nwords // 16, body, 0)


def _zero_mat_ref(ref, nrows, ncols):
    zeros = jnp.zeros((16,), jnp.float32)

    def body(r, _):
        row = ref.at[r]
        for v in range(ncols // 16):
            row[pl.ds(v * 16, 16)] = zeros
        return 0

    lax.fori_loop(0, nrows, body, 0)


def _deg_body(dst_hbm, ew_hbm, deg_out, idx_v, upd_v, stage_v, deg_sp, sem):
    c = lax.axis_index("c")
    s = lax.axis_index("s")
    row0 = s * ROWS_PER_TILE

    _zero_vec_ref(stage_v, ROWS_PER_TILE)
    pltpu.sync_copy(stage_v, deg_sp.at[pl.ds(row0, ROWS_PER_TILE)])

    t0 = c * NS + s
    pltpu.sync_copy(dst_hbm.at[t0], idx_v)
    pltpu.sync_copy(ew_hbm.at[t0], upd_v)
    plsc.subcore_barrier()

    def batch(j, _):
        pltpu.sync_copy(upd_v.at[j], deg_sp.at[idx_v.at[j]], add=True)
        return 0

    lax.fori_loop(0, NBATCH, batch, 0)
    plsc.subcore_barrier()

    pltpu.sync_copy(deg_sp.at[pl.ds(row0, ROWS_PER_TILE)], stage_v)
    pltpu.sync_copy(stage_v, deg_out.at[c, pl.ds(row0, ROWS_PER_TILE)])


@jax.jit
def _deg_call(dst2, ew2):
    k = pl.kernel(
        _deg_body,
        out_type=jax.ShapeDtypeStruct((NC, NPAD), jnp.float32),
        mesh=plsc.VectorSubcoreMesh(**_MESH),
        scratch_types=[
            pltpu.VMEM((NBATCH, BATCH), jnp.int32),
            pltpu.VMEM((NBATCH, BATCH), jnp.float32),
            pltpu.VMEM((ROWS_PER_TILE,), jnp.float32),
            pltpu.VMEM_SHARED((NPAD,), jnp.float32),
            pltpu.SemaphoreType.DMA,
        ],
    )
    return k(dst2, ew2)


_DNUMS = lax.GatherDimensionNumbers(
    offset_dims=(), collapsed_slice_dims=(0,), start_index_map=(0,))


def _agg_body(y_hbm, src_hbm, dst_hbm, ew_hbm, out_hbm,
              sb0, sb1, sb2, db0, db1, db2, eb0, eb1, eb2, g0, g1, g2,
              acc_sp,
              gs0, gs1, gs2, cs0, cs1, cs2, xs0, xs1, xs2,
              is0, is1, is2, es0, es1, es2, xs20, xs21, xs22,
              is20, is21, is22, es20, es21, es22):
    c = lax.axis_index("c")
    s = lax.axis_index("s")
    row0 = s * ROWS_PER_TILE
    t0 = c * NS + s
    ebase = t0 * EDGES_PER_TILE

    SB, XS = (sb0, sb1, sb2), (xs0, xs1, xs2)
    DB, IS = (db0, db1, db2), (is0, is1, is2)
    EB, ES = (eb0, eb1, eb2), (es0, es1, es2)
    G, GS = (g0, g1, g2), (gs0, gs1, gs2)
    CS = (cs0, cs1, cs2)
    XS2 = (xs20, xs21, xs22)
    IS2 = (is20, is21, is22)
    ES2 = (es20, es21, es22)

    _zero_mat_ref(g0, BATCH, D)
    for f in range(ROWS_PER_TILE // BATCH):
        pltpu.sync_copy(g0, acc_sp.at[pl.ds(row0 + f * BATCH, BATCH)])
    plsc.subcore_barrier()

    def edge_slice(hbm, j):
        return hbm.at[pl.ds(ebase + j * BATCH, BATCH)]

    def start_sidx(j, m):
        pltpu.make_async_copy(edge_slice(src_hbm, j), SB[m], XS[m]).start()

    def wait_sidx(j, m):
        pltpu.make_async_copy(edge_slice(src_hbm, j), SB[m], XS[m]).wait()

    def start_didx(j, m):
        pltpu.make_async_copy(edge_slice(dst_hbm, j), DB[m], IS[m]).start()

    def wait_didx(j, m):
        pltpu.make_async_copy(edge_slice(dst_hbm, j), DB[m], IS[m]).wait()

    def start_ew(j, m):
        pltpu.make_async_copy(edge_slice(ew_hbm, j), EB[m], ES[m]).start()

    def wait_ew(j, m):
        pltpu.make_async_copy(edge_slice(ew_hbm, j), EB[m], ES[m]).wait()

    GSPLIT = ((0, 24), (24, 24), (48, 16), (64, 16))
    GSEMS = (GS, XS2, IS2, ES2)

    def start_gather(m):
        for q, (o, n) in enumerate(GSPLIT):
            pltpu.make_async_copy(y_hbm.at[SB[m].at[pl.ds(o, n)]],
                                  G[m].at[pl.ds(o, n)], GSEMS[q][m]).start()

    def wait_gather(m):
        for q, (o, n) in enumerate(GSPLIT):
            pltpu.make_async_copy(y_hbm.at[SB[m].at[pl.ds(o, n)]],
                                  G[m].at[pl.ds(o, n)], GSEMS[q][m]).wait()

    def start_scatter(m):
        pltpu.make_async_copy(G[m], acc_sp.at[DB[m]], CS[m]).start(add=True)

    def wait_scatter(m):
        pltpu.make_async_copy(G[m], acc_sp.at[DB[m]], CS[m]).wait()

    def scale(j, k):
        wait_ew(j, k)
        gbuf, ebuf = G[k], EB[k]

        def group(g, _):
            ew16 = ebuf[pl.ds(g * 16, 16)]
            for l in range(16):
                bc = lax.gather(ew16, jnp.full((16, 1), l, jnp.int32),
                                _DNUMS, slice_sizes=(1,),
                                mode=lax.GatherScatterMode.PROMISE_IN_BOUNDS)
                gr = gbuf.at[g * 16 + l]
                for v in range(D // 16):
                    gr[pl.ds(v * 16, 16)] = gr[pl.ds(v * 16, 16)] * bc
            return 0

        lax.fori_loop(0, BATCH // 16, group, 0)

    def position(j, k, wait_sc=True, sidx2=True, nxt=True):
        k1, k2 = (k + 1) % 3, (k + 2) % 3
        wait_gather(k)
        if wait_sc:
            wait_scatter(k1)
        if sidx2:
            start_sidx(j + 2, k2)
        if nxt:
            start_didx(j + 1, k1)
            start_ew(j + 1, k1)
            wait_sidx(j + 1, k1)
            start_gather(k1)
        scale(j, k)
        wait_didx(j, k)
        start_scatter(k)

    start_sidx(0, 0)
    start_sidx(1, 1)
    start_didx(0, 0)
    start_ew(0, 0)
    wait_sidx(0, 0)
    start_gather(0)

    position(0, 0, wait_sc=False)
    position(1, 1, wait_sc=False)

    def triple(t, _):
        j0 = 2 + 3 * t
        position(j0, 2)
        position(j0 + 1, 0)
        position(j0 + 2, 1)
        return 0

    lax.fori_loop(0, (NBATCH - 5) // 3, triple, 0)

    position(NBATCH - 3, 2)
    position(NBATCH - 2, 0, sidx2=False)
    position(NBATCH - 1, 1, sidx2=False, nxt=False)
    wait_scatter(0)
    wait_scatter(1)

    plsc.subcore_barrier()

    for f in range(ROWS_PER_TILE // BATCH):
        r = row0 + f * BATCH
        pltpu.sync_copy(acc_sp.at[pl.ds(r, BATCH)], g0)
        pltpu.sync_copy(g0, out_hbm.at[c, pl.ds(r, BATCH)])


@jax.jit
def _agg_call(y, src, dst, ew):
    k = pl.kernel(
        _agg_body,
        out_type=jax.ShapeDtypeStruct((NC, NPAD, D), jnp.float32),
        mesh=plsc.VectorSubcoreMesh(**_MESH),
        scratch_types=(
            [pltpu.VMEM((BATCH,), jnp.int32) for _ in range(3)]
            + [pltpu.VMEM((BATCH,), jnp.int32) for _ in range(3)]
            + [pltpu.VMEM((BATCH,), jnp.float32) for _ in range(3)]
            + [pltpu.VMEM((BATCH, D), jnp.float32) for _ in range(3)]
            + [pltpu.VMEM_SHARED((NPAD, D), jnp.float32)]
            + [pltpu.SemaphoreType.DMA for _ in range(24)]
        ),
    )
    return k(y, src, dst, ew)


def _dinv_from(deg2):
    deg = deg2[0, :N] + deg2[1, :N] + 1.0
    safe = jnp.where(deg > 0.0, deg, 1.0)
    return jnp.where(deg > 0.0, lax.rsqrt(safe), 0.0)[:, None]


def _tc0_body(x_ref, w_ref, deg_ref, y_ref, dinv_ref):
    dinv = _dinv_from(deg_ref[...])
    xw = jnp.dot(x_ref[...], w_ref[...], preferred_element_type=jnp.float32)
    y_ref[...] = xw * dinv
    dinv_ref[...] = dinv


@jax.jit
def _tc0_call(x, W1, deg2):
    return pl.pallas_call(
        _tc0_body,
        out_shape=(
            jax.ShapeDtypeStruct((N, D), jnp.float32),
            jax.ShapeDtypeStruct((N, 1), jnp.float32),
        ),
    )(x, W1, deg2)


def _mix_body(agg_ref, y_ref, dinv_ref, b_ref, tp_ref, w_ref,
              temp_ref, ynext_ref):
    agg = agg_ref[0, :N, :] + agg_ref[1, :N, :]
    dinv = dinv_ref[...]
    out = dinv * (agg + y_ref[...]) + b_ref[...]
    temp = (1.0 - PRESERVE_C) * out + PRESERVE_C * tp_ref[...]
    temp_ref[...] = temp
    ynext_ref[...] = jnp.dot(temp, w_ref[...],
                             preferred_element_type=jnp.float32) * dinv


@jax.jit
def _mix_call(agg2, y, dinv, b, temp_prev, Wn):
    return pl.pallas_call(
        _mix_body,
        out_shape=(
            jax.ShapeDtypeStruct((N, D), jnp.float32),
            jax.ShapeDtypeStruct((N, D), jnp.float32),
        ),
    )(agg2, y, dinv, b, temp_prev, Wn)


def _fin_body(agg_ref, y_ref, dinv_ref, b_ref, tp_ref, temp_ref):
    agg = agg_ref[0, :N, :] + agg_ref[1, :N, :]
    out = dinv_ref[...] * (agg + y_ref[...]) + b_ref[...]
    temp_ref[...] = (1.0 - PRESERVE_C) * out + PRESERVE_C * tp_ref[...]


@jax.jit
def _fin_call(agg2, y, dinv, b, temp_prev):
    return pl.pallas_call(
        _fin_body,
        out_shape=jax.ShapeDtypeStruct((N, D), jnp.float32),
    )(agg2, y, dinv, b, temp_prev)


def kernel(skill_embed, adj_list, edge_attr, W1, b1, W2, b2, W3, b3):
    src = adj_list[0]
    dst = adj_list[1]
    dst3 = adj_list[1].reshape(NC * NS, NBATCH, BATCH)
    ew3 = edge_attr.reshape(NC * NS, NBATCH, BATCH)

    deg2 = _deg_call(dst3, ew3)
    y1, dinv = _tc0_call(skill_embed, W1, deg2)

    agg1 = _agg_call(y1, src, dst, edge_attr)
    temp1, y2 = _mix_call(agg1, y1, dinv, b1.reshape(1, D), skill_embed, W2)

    agg2 = _agg_call(y2, src, dst, edge_attr)
    temp2, y3 = _mix_call(agg2, y2, dinv, b2.reshape(1, D), temp1, W3)

    agg3 = _agg_call(y3, src, dst, edge_attr)
    return _fin_call(agg3, y3, dinv, b3.reshape(1, D), temp2)


# speedup vs baseline: 1.0045x; 1.0045x over previous
import functools

import jax
import jax.numpy as jnp
from jax import lax
from jax.experimental import pallas as pl
from jax.experimental.pallas import tpu as pltpu
from jax.experimental.pallas import tpu_sc as plsc

N = 10000
E = 320000
D = 128
NPAD = 10240
NC = 2
NS = 16
EDGES_PER_SC = E // NC
EDGES_PER_TILE = EDGES_PER_SC // NS
BATCH = 80
NBATCH = EDGES_PER_TILE // BATCH
ROWS_PER_TILE = NPAD // NS
FLUSH_ROWS = 128
PRESERVE_C = 0.1

_MESH = dict(core_axis_name="c", subcore_axis_name="s")


def _zero_vec_ref(ref, nwords):
    zeros = jnp.zeros((16,), jnp.float32)

    def body(i, _):
        ref[pl.ds(i * 16, 16)] = zeros
        return 0

    lax.fori_loop(0, nwords // 16, body, 0)


def _zero_mat_ref(ref, nrows, ncols):
    zeros = jnp.zeros((16,), jnp.float32)

    def body(r, _):
        row = ref.at[r]
        for v in range(ncols // 16):
            row[pl.ds(v * 16, 16)] = zeros
        return 0

    lax.fori_loop(0, nrows, body, 0)


def _deg_body(dst_hbm, ew_hbm, deg_out, idx_v, upd_v, stage_v, deg_sp, sem):
    c = lax.axis_index("c")
    s = lax.axis_index("s")
    row0 = s * ROWS_PER_TILE

    _zero_vec_ref(stage_v, ROWS_PER_TILE)
    pltpu.sync_copy(stage_v, deg_sp.at[pl.ds(row0, ROWS_PER_TILE)])

    t0 = c * NS + s
    pltpu.sync_copy(dst_hbm.at[t0], idx_v)
    pltpu.sync_copy(ew_hbm.at[t0], upd_v)
    plsc.subcore_barrier()

    def batch(j, _):
        pltpu.sync_copy(upd_v.at[j], deg_sp.at[idx_v.at[j]], add=True)
        return 0

    lax.fori_loop(0, NBATCH, batch, 0)
    plsc.subcore_barrier()

    pltpu.sync_copy(deg_sp.at[pl.ds(row0, ROWS_PER_TILE)], stage_v)
    pltpu.sync_copy(stage_v, deg_out.at[c, pl.ds(row0, ROWS_PER_TILE)])


@jax.jit
def _deg_call(dst2, ew2):
    k = pl.kernel(
        _deg_body,
        out_type=jax.ShapeDtypeStruct((NC, NPAD), jnp.float32),
        mesh=plsc.VectorSubcoreMesh(**_MESH),
        scratch_types=[
            pltpu.VMEM((NBATCH, BATCH), jnp.int32),
            pltpu.VMEM((NBATCH, BATCH), jnp.float32),
            pltpu.VMEM((ROWS_PER_TILE,), jnp.float32),
            pltpu.VMEM_SHARED((NPAD,), jnp.float32),
            pltpu.SemaphoreType.DMA,
        ],
    )
    return k(dst2, ew2)


_DNUMS = lax.GatherDimensionNumbers(
    offset_dims=(), collapsed_slice_dims=(0,), start_index_map=(0,))


def _agg_body(y_hbm, src_hbm, dst_hbm, ew_hbm, out_hbm,
              sb0, sb1, sb2, db0, db1, db2, eb0, eb1, eb2, g0, g1, g2,
              acc_sp,
              gs0, gs1, gs2, cs0, cs1, cs2, xs0, xs1, xs2,
              is0, is1, is2, es0, es1, es2, xs20, xs21, xs22,
              is20, is21, is22, es20, es21, es22):
    c = lax.axis_index("c")
    s = lax.axis_index("s")
    row0 = s * ROWS_PER_TILE
    t0 = c * NS + s
    ebase = t0 * EDGES_PER_TILE

    SB, XS = (sb0, sb1, sb2), (xs0, xs1, xs2)
    DB, IS = (db0, db1, db2), (is0, is1, is2)
    EB, ES = (eb0, eb1, eb2), (es0, es1, es2)
    G, GS = (g0, g1, g2), (gs0, gs1, gs2)
    CS = (cs0, cs1, cs2)
    XS2 = (xs20, xs21, xs22)
    IS2 = (is20, is21, is22)
    ES2 = (es20, es21, es22)

    _zero_mat_ref(g0, BATCH, D)
    for f in range(ROWS_PER_TILE // BATCH):
        pltpu.sync_copy(g0, acc_sp.at[pl.ds(row0 + f * BATCH, BATCH)])
    plsc.subcore_barrier()

    def edge_slice(hbm, j):
        return hbm.at[pl.ds(ebase + j * BATCH, BATCH)]

    def start_sidx(j, m):
        pltpu.make_async_copy(edge_slice(src_hbm, j), SB[m], XS[m]).start()

    def wait_sidx(j, m):
        pltpu.make_async_copy(edge_slice(src_hbm, j), SB[m], XS[m]).wait()

    def start_didx(j, m):
        pltpu.make_async_copy(edge_slice(dst_hbm, j), DB[m], IS[m]).start()

    def wait_didx(j, m):
        pltpu.make_async_copy(edge_slice(dst_hbm, j), DB[m], IS[m]).wait()

    def start_ew(j, m):
        pltpu.make_async_copy(edge_slice(ew_hbm, j), EB[m], ES[m]).start()

    def wait_ew(j, m):
        pltpu.make_async_copy(edge_slice(ew_hbm, j), EB[m], ES[m]).wait()

    GSPLIT = ((0, 32), (32, 32), (64, 16))
    GSEMS = (GS, XS2, IS2)

    def start_gather(m):
        for q, (o, n) in enumerate(GSPLIT):
            pltpu.make_async_copy(y_hbm.at[SB[m].at[pl.ds(o, n)]],
                                  G[m].at[pl.ds(o, n)], GSEMS[q][m]).start()

    def wait_gather(m):
        for q, (o, n) in enumerate(GSPLIT):
            pltpu.make_async_copy(y_hbm.at[SB[m].at[pl.ds(o, n)]],
                                  G[m].at[pl.ds(o, n)], GSEMS[q][m]).wait()

    def start_scatter(m):
        pltpu.make_async_copy(G[m], acc_sp.at[DB[m]], CS[m]).start(add=True)

    def wait_scatter(m):
        pltpu.make_async_copy(G[m], acc_sp.at[DB[m]], CS[m]).wait()

    def scale(j, k):
        wait_ew(j, k)
        gbuf, ebuf = G[k], EB[k]

        def group(g, _):
            ew16 = ebuf[pl.ds(g * 16, 16)]
            for l in range(16):
                bc = lax.gather(ew16, jnp.full((16, 1), l, jnp.int32),
                                _DNUMS, slice_sizes=(1,),
                                mode=lax.GatherScatterMode.PROMISE_IN_BOUNDS)
                gr = gbuf.at[g * 16 + l]
                for v in range(D // 16):
                    gr[pl.ds(v * 16, 16)] = gr[pl.ds(v * 16, 16)] * bc
            return 0

        lax.fori_loop(0, BATCH // 16, group, 0)

    def position(j, k, wait_sc=True, sidx2=True, nxt=True):
        k1, k2 = (k + 1) % 3, (k + 2) % 3
        wait_gather(k)
        if wait_sc:
            wait_scatter(k1)
        if sidx2:
            start_sidx(j + 2, k2)
        if nxt:
            start_didx(j + 1, k1)
            start_ew(j + 1, k1)
            wait_sidx(j + 1, k1)
            start_gather(k1)
        scale(j, k)
        wait_didx(j, k)
        start_scatter(k)

    start_sidx(0, 0)
    start_sidx(1, 1)
    start_didx(0, 0)
    start_ew(0, 0)
    wait_sidx(0, 0)
    start_gather(0)

    position(0, 0, wait_sc=False)
    position(1, 1, wait_sc=False)

    def triple(t, _):
        j0 = 2 + 3 * t
        position(j0, 2)
        position(j0 + 1, 0)
        position(j0 + 2, 1)
        return 0

    lax.fori_loop(0, (NBATCH - 5) // 3, triple, 0)

    position(NBATCH - 3, 2)
    position(NBATCH - 2, 0, sidx2=False)
    position(NBATCH - 1, 1, sidx2=False, nxt=False)
    wait_scatter(0)
    wait_scatter(1)

    plsc.subcore_barrier()

    for f in range(ROWS_PER_TILE // BATCH):
        r = row0 + f * BATCH
        pltpu.sync_copy(acc_sp.at[pl.ds(r, BATCH)], g0)
        pltpu.sync_copy(g0, out_hbm.at[c, pl.ds(r, BATCH)])


@jax.jit
def _agg_call(y, src, dst, ew):
    k = pl.kernel(
        _agg_body,
        out_type=jax.ShapeDtypeStruct((NC, NPAD, D), jnp.float32),
        mesh=plsc.VectorSubcoreMesh(**_MESH),
        scratch_types=(
            [pltpu.VMEM((BATCH,), jnp.int32) for _ in range(3)]
            + [pltpu.VMEM((BATCH,), jnp.int32) for _ in range(3)]
            + [pltpu.VMEM((BATCH,), jnp.float32) for _ in range(3)]
            + [pltpu.VMEM((BATCH, D), jnp.float32) for _ in range(3)]
            + [pltpu.VMEM_SHARED((NPAD, D), jnp.float32)]
            + [pltpu.SemaphoreType.DMA for _ in range(24)]
        ),
    )
    return k(y, src, dst, ew)


def _dinv_from(deg2):
    deg = deg2[0, :N] + deg2[1, :N] + 1.0
    safe = jnp.where(deg > 0.0, deg, 1.0)
    return jnp.where(deg > 0.0, lax.rsqrt(safe), 0.0)[:, None]


def _tc0_body(x_ref, w_ref, deg_ref, y_ref, dinv_ref):
    dinv = _dinv_from(deg_ref[...])
    xw = jnp.dot(x_ref[...], w_ref[...], preferred_element_type=jnp.float32)
    y_ref[...] = xw * dinv
    dinv_ref[...] = dinv


@jax.jit
def _tc0_call(x, W1, deg2):
    return pl.pallas_call(
        _tc0_body,
        out_shape=(
            jax.ShapeDtypeStruct((N, D), jnp.float32),
            jax.ShapeDtypeStruct((N, 1), jnp.float32),
        ),
    )(x, W1, deg2)


def _mix_body(agg_ref, y_ref, dinv_ref, b_ref, tp_ref, w_ref,
              temp_ref, ynext_ref):
    agg = agg_ref[0, :N, :] + agg_ref[1, :N, :]
    dinv = dinv_ref[...]
    out = dinv * (agg + y_ref[...]) + b_ref[...]
    temp = (1.0 - PRESERVE_C) * out + PRESERVE_C * tp_ref[...]
    temp_ref[...] = temp
    ynext_ref[...] = jnp.dot(temp, w_ref[...],
                             preferred_element_type=jnp.float32) * dinv


@jax.jit
def _mix_call(agg2, y, dinv, b, temp_prev, Wn):
    return pl.pallas_call(
        _mix_body,
        out_shape=(
            jax.ShapeDtypeStruct((N, D), jnp.float32),
            jax.ShapeDtypeStruct((N, D), jnp.float32),
        ),
    )(agg2, y, dinv, b, temp_prev, Wn)


def _fin_body(agg_ref, y_ref, dinv_ref, b_ref, tp_ref, temp_ref):
    agg = agg_ref[0, :N, :] + agg_ref[1, :N, :]
    out = dinv_ref[...] * (agg + y_ref[...]) + b_ref[...]
    temp_ref[...] = (1.0 - PRESERVE_C) * out + PRESERVE_C * tp_ref[...]


@jax.jit
def _fin_call(agg2, y, dinv, b, temp_prev):
    return pl.pallas_call(
        _fin_body,
        out_shape=jax.ShapeDtypeStruct((N, D), jnp.float32),
    )(agg2, y, dinv, b, temp_prev)


def kernel(skill_embed, adj_list, edge_attr, W1, b1, W2, b2, W3, b3):
    src = adj_list[0]
    dst = adj_list[1]
    dst3 = adj_list[1].reshape(NC * NS, NBATCH, BATCH)
    ew3 = edge_attr.reshape(NC * NS, NBATCH, BATCH)

    deg2 = _deg_call(dst3, ew3)
    y1, dinv = _tc0_call(skill_embed, W1, deg2)

    agg1 = _agg_call(y1, src, dst, edge_attr)
    temp1, y2 = _mix_call(agg1, y1, dinv, b1.reshape(1, D), skill_embed, W2)

    agg2 = _agg_call(y2, src, dst, edge_attr)
    temp2, y3 = _mix_call(agg2, y2, dinv, b2.reshape(1, D), temp1, W3)

    agg3 = _agg_call(y3, src, dst, edge_attr)
    return _fin_call(agg3, y3, dinv, b3.reshape(1, D), temp2)


# speedup vs baseline: 1.0962x; 1.0913x over previous
import functools

import jax
import jax.numpy as jnp
from jax import lax
from jax.experimental import pallas as pl
from jax.experimental.pallas import tpu as pltpu
from jax.experimental.pallas import tpu_sc as plsc

N = 10000
E = 320000
D = 128
NPAD = 10240
NC = 2
NS = 16
EDGES_PER_SC = E // NC
EDGES_PER_TILE = EDGES_PER_SC // NS
BATCH = 80
NBATCH = EDGES_PER_TILE // BATCH
ROWS_PER_TILE = NPAD // NS
FLUSH_ROWS = 128
PRESERVE_C = 0.1

_MESH = dict(core_axis_name="c", subcore_axis_name="s")


def _zero_vec_ref(ref, nwords):
    zeros = jnp.zeros((16,), jnp.float32)

    def body(i, _):
        ref[pl.ds(i * 16, 16)] = zeros
        return 0

    lax.fori_loop(0, nwords // 16, body, 0)


def _zero_mat_ref(ref, nrows, ncols):
    zeros = jnp.zeros((16,), jnp.float32)

    def body(r, _):
        row = ref.at[r]
        for v in range(ncols // 16):
            row[pl.ds(v * 16, 16)] = zeros
        return 0

    lax.fori_loop(0, nrows, body, 0)


def _deg_body(dst_hbm, ew_hbm, deg_out, idx_v, upd_v, stage_v, deg_sp, sem):
    c = lax.axis_index("c")
    s = lax.axis_index("s")
    row0 = s * ROWS_PER_TILE

    _zero_vec_ref(stage_v, ROWS_PER_TILE)
    pltpu.sync_copy(stage_v, deg_sp.at[pl.ds(row0, ROWS_PER_TILE)])

    t0 = c * NS + s
    pltpu.sync_copy(dst_hbm.at[t0], idx_v)
    pltpu.sync_copy(ew_hbm.at[t0], upd_v)
    plsc.subcore_barrier()

    def batch(j, _):
        pltpu.sync_copy(upd_v.at[j], deg_sp.at[idx_v.at[j]], add=True)
        return 0

    lax.fori_loop(0, NBATCH, batch, 0)
    plsc.subcore_barrier()

    pltpu.sync_copy(deg_sp.at[pl.ds(row0, ROWS_PER_TILE)], stage_v)
    pltpu.sync_copy(stage_v, deg_out.at[c, pl.ds(row0, ROWS_PER_TILE)])


@jax.jit
def _deg_call(dst2, ew2):
    k = pl.kernel(
        _deg_body,
        out_type=jax.ShapeDtypeStruct((NC, NPAD), jnp.float32),
        mesh=plsc.VectorSubcoreMesh(**_MESH),
        scratch_types=[
            pltpu.VMEM((NBATCH, BATCH), jnp.int32),
            pltpu.VMEM((NBATCH, BATCH), jnp.float32),
            pltpu.VMEM((ROWS_PER_TILE,), jnp.float32),
            pltpu.VMEM_SHARED((NPAD,), jnp.float32),
            pltpu.SemaphoreType.DMA,
        ],
    )
    return k(dst2, ew2)


_DNUMS = lax.GatherDimensionNumbers(
    offset_dims=(), collapsed_slice_dims=(0,), start_index_map=(0,))


def _agg_body(y_hbm, src_hbm, dst_hbm, ew_hbm, out_hbm,
              sb0, sb1, sb2, db0, db1, db2, eb0, eb1, eb2, g0, g1, g2,
              acc_sp,
              gs0, gs1, gs2, cs0, cs1, cs2, xs0, xs1, xs2,
              is0, is1, is2, es0, es1, es2, xs20, xs21, xs22,
              is20, is21, is22, es20, es21, es22):
    c = lax.axis_index("c")
    s = lax.axis_index("s")
    row0 = s * ROWS_PER_TILE
    t0 = c * NS + s
    ebase = t0 * EDGES_PER_TILE

    SB, XS = (sb0, sb1, sb2), (xs0, xs1, xs2)
    DB, IS = (db0, db1, db2), (is0, is1, is2)
    EB, ES = (eb0, eb1, eb2), (es0, es1, es2)
    G, GS = (g0, g1, g2), (gs0, gs1, gs2)
    CS = (cs0, cs1, cs2)
    XS2 = (xs20, xs21, xs22)
    IS2 = (is20, is21, is22)
    ES2 = (es20, es21, es22)

    _zero_mat_ref(g0, BATCH, D)
    for f in range(ROWS_PER_TILE // BATCH):
        pltpu.sync_copy(g0, acc_sp.at[pl.ds(row0 + f * BATCH, BATCH)])
    plsc.subcore_barrier()

    def edge_slice(hbm, j):
        return hbm.at[pl.ds(ebase + j * BATCH, BATCH)]

    def start_sidx(j, m):
        pltpu.make_async_copy(edge_slice(src_hbm, j), SB[m], XS[m]).start()

    def wait_sidx(j, m):
        pltpu.make_async_copy(edge_slice(src_hbm, j), SB[m], XS[m]).wait()

    def start_didx(j, m):
        pltpu.make_async_copy(edge_slice(dst_hbm, j), DB[m], IS[m]).start()

    def wait_didx(j, m):
        pltpu.make_async_copy(edge_slice(dst_hbm, j), DB[m], IS[m]).wait()

    def start_ew(j, m):
        pltpu.make_async_copy(edge_slice(ew_hbm, j), EB[m], ES[m]).start()

    def wait_ew(j, m):
        pltpu.make_async_copy(edge_slice(ew_hbm, j), EB[m], ES[m]).wait()

    GSPLIT = ((0, 40), (40, 40))
    GSEMS = (GS, XS2)

    def start_gather(m):
        for q, (o, n) in enumerate(GSPLIT):
            pltpu.make_async_copy(y_hbm.at[SB[m].at[pl.ds(o, n)]],
                                  G[m].at[pl.ds(o, n)], GSEMS[q][m]).start()

    def wait_gather(m):
        for q, (o, n) in enumerate(GSPLIT):
            pltpu.make_async_copy(y_hbm.at[SB[m].at[pl.ds(o, n)]],
                                  G[m].at[pl.ds(o, n)], GSEMS[q][m]).wait()

    def start_scatter(m):
        pltpu.make_async_copy(G[m], acc_sp.at[DB[m]], CS[m]).start(add=True)

    def wait_scatter(m):
        pltpu.make_async_copy(G[m], acc_sp.at[DB[m]], CS[m]).wait()

    def scale(j, k):
        wait_ew(j, k)
        gbuf, ebuf = G[k], EB[k]

        def group(g, _):
            ew16 = ebuf[pl.ds(g * 16, 16)]
            for l in range(16):
                bc = lax.gather(ew16, jnp.full((16, 1), l, jnp.int32),
                                _DNUMS, slice_sizes=(1,),
                                mode=lax.GatherScatterMode.PROMISE_IN_BOUNDS)
                gr = gbuf.at[g * 16 + l]
                for v in range(D // 16):
                    gr[pl.ds(v * 16, 16)] = gr[pl.ds(v * 16, 16)] * bc
            return 0

        lax.fori_loop(0, BATCH // 16, group, 0)

    def position(j, k, wait_sc=True, sidx2=True, nxt=True):
        k1, k2 = (k + 1) % 3, (k + 2) % 3
        if nxt:
            if wait_sc:
                wait_scatter(k1)
            wait_sidx(j + 1, k1)
            start_gather(k1)
        wait_gather(k)
        if sidx2:
            start_sidx(j + 2, k2)
        if nxt:
            start_didx(j + 1, k1)
            start_ew(j + 1, k1)
        scale(j, k)
        wait_didx(j, k)
        start_scatter(k)

    start_sidx(0, 0)
    start_sidx(1, 1)
    start_didx(0, 0)
    start_ew(0, 0)
    wait_sidx(0, 0)
    start_gather(0)

    position(0, 0, wait_sc=False)
    position(1, 1, wait_sc=False)

    def triple(t, _):
        j0 = 2 + 3 * t
        position(j0, 2)
        position(j0 + 1, 0)
        position(j0 + 2, 1)
        return 0

    lax.fori_loop(0, (NBATCH - 5) // 3, triple, 0)

    position(NBATCH - 3, 2)
    position(NBATCH - 2, 0, sidx2=False)
    position(NBATCH - 1, 1, sidx2=False, nxt=False)
    wait_scatter(0)
    wait_scatter(1)

    plsc.subcore_barrier()

    for f in range(ROWS_PER_TILE // BATCH):
        r = row0 + f * BATCH
        pltpu.sync_copy(acc_sp.at[pl.ds(r, BATCH)], g0)
        pltpu.sync_copy(g0, out_hbm.at[c, pl.ds(r, BATCH)])


@jax.jit
def _agg_call(y, src, dst, ew):
    k = pl.kernel(
        _agg_body,
        out_type=jax.ShapeDtypeStruct((NC, NPAD, D), jnp.float32),
        mesh=plsc.VectorSubcoreMesh(**_MESH),
        scratch_types=(
            [pltpu.VMEM((BATCH,), jnp.int32) for _ in range(3)]
            + [pltpu.VMEM((BATCH,), jnp.int32) for _ in range(3)]
            + [pltpu.VMEM((BATCH,), jnp.float32) for _ in range(3)]
            + [pltpu.VMEM((BATCH, D), jnp.float32) for _ in range(3)]
            + [pltpu.VMEM_SHARED((NPAD, D), jnp.float32)]
            + [pltpu.SemaphoreType.DMA for _ in range(24)]
        ),
    )
    return k(y, src, dst, ew)


def _dinv_from(deg2):
    deg = deg2[0, :N] + deg2[1, :N] + 1.0
    safe = jnp.where(deg > 0.0, deg, 1.0)
    return jnp.where(deg > 0.0, lax.rsqrt(safe), 0.0)[:, None]


def _tc0_body(x_ref, w_ref, deg_ref, y_ref, dinv_ref):
    dinv = _dinv_from(deg_ref[...])
    xw = jnp.dot(x_ref[...], w_ref[...], preferred_element_type=jnp.float32)
    y_ref[...] = xw * dinv
    dinv_ref[...] = dinv


@jax.jit
def _tc0_call(x, W1, deg2):
    return pl.pallas_call(
        _tc0_body,
        out_shape=(
            jax.ShapeDtypeStruct((N, D), jnp.float32),
            jax.ShapeDtypeStruct((N, 1), jnp.float32),
        ),
    )(x, W1, deg2)


def _mix_body(agg_ref, y_ref, dinv_ref, b_ref, tp_ref, w_ref,
              temp_ref, ynext_ref):
    agg = agg_ref[0, :N, :] + agg_ref[1, :N, :]
    dinv = dinv_ref[...]
    out = dinv * (agg + y_ref[...]) + b_ref[...]
    temp = (1.0 - PRESERVE_C) * out + PRESERVE_C * tp_ref[...]
    temp_ref[...] = temp
    ynext_ref[...] = jnp.dot(temp, w_ref[...],
                             preferred_element_type=jnp.float32) * dinv


@jax.jit
def _mix_call(agg2, y, dinv, b, temp_prev, Wn):
    return pl.pallas_call(
        _mix_body,
        out_shape=(
            jax.ShapeDtypeStruct((N, D), jnp.float32),
            jax.ShapeDtypeStruct((N, D), jnp.float32),
        ),
    )(agg2, y, dinv, b, temp_prev, Wn)


def _fin_body(agg_ref, y_ref, dinv_ref, b_ref, tp_ref, temp_ref):
    agg = agg_ref[0, :N, :] + agg_ref[1, :N, :]
    out = dinv_ref[...] * (agg + y_ref[...]) + b_ref[...]
    temp_ref[...] = (1.0 - PRESERVE_C) * out + PRESERVE_C * tp_ref[...]


@jax.jit
def _fin_call(agg2, y, dinv, b, temp_prev):
    return pl.pallas_call(
        _fin_body,
        out_shape=jax.ShapeDtypeStruct((N, D), jnp.float32),
    )(agg2, y, dinv, b, temp_prev)


def kernel(skill_embed, adj_list, edge_attr, W1, b1, W2, b2, W3, b3):
    src = adj_list[0]
    dst = adj_list[1]
    dst3 = adj_list[1].reshape(NC * NS, NBATCH, BATCH)
    ew3 = edge_attr.reshape(NC * NS, NBATCH, BATCH)

    deg2 = _deg_call(dst3, ew3)
    y1, dinv = _tc0_call(skill_embed, W1, deg2)

    agg1 = _agg_call(y1, src, dst, edge_attr)
    temp1, y2 = _mix_call(agg1, y1, dinv, b1.reshape(1, D), skill_embed, W2)

    agg2 = _agg_call(y2, src, dst, edge_attr)
    temp2, y3 = _mix_call(agg2, y2, dinv, b2.reshape(1, D), temp1, W3)

    agg3 = _agg_call(y3, src, dst, edge_attr)
    return _fin_call(agg3, y3, dinv, b3.reshape(1, D), temp2)


# speedup vs baseline: 1.1000x; 1.0035x over previous
import functools

import jax
import jax.numpy as jnp
from jax import lax
from jax.experimental import pallas as pl
from jax.experimental.pallas import tpu as pltpu
from jax.experimental.pallas import tpu_sc as plsc

N = 10000
E = 320000
D = 128
NPAD = 10240
NC = 2
NS = 16
EDGES_PER_SC = E // NC
EDGES_PER_TILE = EDGES_PER_SC // NS
BATCH = 80
NBATCH = EDGES_PER_TILE // BATCH
ROWS_PER_TILE = NPAD // NS
FLUSH_ROWS = 128
PRESERVE_C = 0.1

_MESH = dict(core_axis_name="c", subcore_axis_name="s")


def _zero_vec_ref(ref, nwords):
    zeros = jnp.zeros((16,), jnp.float32)

    def body(i, _):
        ref[pl.ds(i * 16, 16)] = zeros
        return 0

    lax.fori_loop(0, nwords // 16, body, 0)


def _zero_mat_ref(ref, nrows, ncols):
    zeros = jnp.zeros((16,), jnp.float32)

    def body(r, _):
        row = ref.at[r]
        for v in range(ncols // 16):
            row[pl.ds(v * 16, 16)] = zeros
        return 0

    lax.fori_loop(0, nrows, body, 0)


def _deg_body(dst_hbm, ew_hbm, deg_out, idx_v, upd_v, stage_v, deg_sp, sem):
    c = lax.axis_index("c")
    s = lax.axis_index("s")
    row0 = s * ROWS_PER_TILE

    _zero_vec_ref(stage_v, ROWS_PER_TILE)
    pltpu.sync_copy(stage_v, deg_sp.at[pl.ds(row0, ROWS_PER_TILE)])

    t0 = c * NS + s
    pltpu.sync_copy(dst_hbm.at[t0], idx_v)
    pltpu.sync_copy(ew_hbm.at[t0], upd_v)
    plsc.subcore_barrier()

    def batch(j, _):
        pltpu.sync_copy(upd_v.at[j], deg_sp.at[idx_v.at[j]], add=True)
        return 0

    lax.fori_loop(0, NBATCH, batch, 0)
    plsc.subcore_barrier()

    pltpu.sync_copy(deg_sp.at[pl.ds(row0, ROWS_PER_TILE)], stage_v)
    pltpu.sync_copy(stage_v, deg_out.at[c, pl.ds(row0, ROWS_PER_TILE)])


@jax.jit
def _deg_call(dst2, ew2):
    k = pl.kernel(
        _deg_body,
        out_type=jax.ShapeDtypeStruct((NC, NPAD), jnp.float32),
        mesh=plsc.VectorSubcoreMesh(**_MESH),
        scratch_types=[
            pltpu.VMEM((NBATCH, BATCH), jnp.int32),
            pltpu.VMEM((NBATCH, BATCH), jnp.float32),
            pltpu.VMEM((ROWS_PER_TILE,), jnp.float32),
            pltpu.VMEM_SHARED((NPAD,), jnp.float32),
            pltpu.SemaphoreType.DMA,
        ],
    )
    return k(dst2, ew2)


_DNUMS = lax.GatherDimensionNumbers(
    offset_dims=(), collapsed_slice_dims=(0,), start_index_map=(0,))


def _agg_body(y_hbm, src_hbm, dst_hbm, ew_hbm, out_hbm,
              sb0, sb1, sb2, db0, db1, db2, eb0, eb1, eb2, g0, g1, g2,
              acc_sp,
              gs0, gs1, gs2, cs0, cs1, cs2, xs0, xs1, xs2,
              is0, is1, is2, es0, es1, es2, xs20, xs21, xs22,
              is20, is21, is22, es20, es21, es22):
    c = lax.axis_index("c")
    s = lax.axis_index("s")
    row0 = s * ROWS_PER_TILE
    t0 = c * NS + s
    ebase = t0 * EDGES_PER_TILE

    SB, XS = (sb0, sb1, sb2), (xs0, xs1, xs2)
    DB, IS = (db0, db1, db2), (is0, is1, is2)
    EB, ES = (eb0, eb1, eb2), (es0, es1, es2)
    G, GS = (g0, g1, g2), (gs0, gs1, gs2)
    CS = (cs0, cs1, cs2)
    XS2 = (xs20, xs21, xs22)
    IS2 = (is20, is21, is22)
    ES2 = (es20, es21, es22)

    _zero_mat_ref(g0, BATCH, D)
    for f in range(ROWS_PER_TILE // BATCH):
        pltpu.sync_copy(g0, acc_sp.at[pl.ds(row0 + f * BATCH, BATCH)])
    plsc.subcore_barrier()

    def edge_slice(hbm, j):
        return hbm.at[pl.ds(ebase + j * BATCH, BATCH)]

    def start_sidx(j, m):
        pltpu.make_async_copy(edge_slice(src_hbm, j), SB[m], XS[m]).start()

    def wait_sidx(j, m):
        pltpu.make_async_copy(edge_slice(src_hbm, j), SB[m], XS[m]).wait()

    def start_didx(j, m):
        pltpu.make_async_copy(edge_slice(dst_hbm, j), DB[m], IS[m]).start()

    def wait_didx(j, m):
        pltpu.make_async_copy(edge_slice(dst_hbm, j), DB[m], IS[m]).wait()

    def start_ew(j, m):
        pltpu.make_async_copy(edge_slice(ew_hbm, j), EB[m], ES[m]).start()

    def wait_ew(j, m):
        pltpu.make_async_copy(edge_slice(ew_hbm, j), EB[m], ES[m]).wait()

    GSPLIT = ((0, 40), (40, 40))
    GSEMS = (GS, XS2)

    def start_gather(m):
        for q, (o, n) in enumerate(GSPLIT):
            pltpu.make_async_copy(y_hbm.at[SB[m].at[pl.ds(o, n)]],
                                  G[m].at[pl.ds(o, n)], GSEMS[q][m]).start()

    def wait_gather(m):
        for q, (o, n) in enumerate(GSPLIT):
            pltpu.make_async_copy(y_hbm.at[SB[m].at[pl.ds(o, n)]],
                                  G[m].at[pl.ds(o, n)], GSEMS[q][m]).wait()

    def start_scatter(m):
        pltpu.make_async_copy(G[m], acc_sp.at[DB[m]], CS[m]).start(add=True)

    def wait_scatter(m):
        pltpu.make_async_copy(G[m], acc_sp.at[DB[m]], CS[m]).wait()

    def scale(j, k):
        wait_ew(j, k)
        gbuf, ebuf = G[k], EB[k]

        def group(g, _):
            ew16 = ebuf[pl.ds(g * 16, 16)]
            for l in range(16):
                bc = lax.gather(ew16, jnp.full((16, 1), l, jnp.int32),
                                _DNUMS, slice_sizes=(1,),
                                mode=lax.GatherScatterMode.PROMISE_IN_BOUNDS)
                gr = gbuf.at[g * 16 + l]
                for v in range(D // 16):
                    gr[pl.ds(v * 16, 16)] = gr[pl.ds(v * 16, 16)] * bc
            return 0

        lax.fori_loop(0, BATCH // 16, group, 0)

    def position(j, k, wait_sc=True, sidx2=True, nxt=True):
        k1, k2 = (k + 1) % 3, (k + 2) % 3
        if nxt:
            if wait_sc:
                wait_scatter(k1)
            wait_sidx(j + 1, k1)
            start_gather(k1)
        wait_gather(k)
        if sidx2:
            start_sidx(j + 2, k2)
        if nxt:
            start_didx(j + 1, k1)
            start_ew(j + 1, k1)
        scale(j, k)
        wait_didx(j, k)
        start_scatter(k)

    start_sidx(0, 0)
    start_sidx(1, 1)
    start_didx(0, 0)
    start_ew(0, 0)
    wait_sidx(0, 0)
    start_gather(0)

    position(0, 0, wait_sc=False)
    position(1, 1, wait_sc=False)

    def triple(t, _):
        j0 = 2 + 3 * t
        position(j0, 2)
        position(j0 + 1, 0)
        position(j0 + 2, 1)
        return 0

    lax.fori_loop(0, (NBATCH - 5) // 3, triple, 0)

    position(NBATCH - 3, 2)
    position(NBATCH - 2, 0, sidx2=False)
    position(NBATCH - 1, 1, sidx2=False, nxt=False)
    wait_scatter(2)
    wait_scatter(0)
    wait_scatter(1)

    plsc.subcore_barrier()

    for f in range(ROWS_PER_TILE // BATCH):
        r = row0 + f * BATCH
        pltpu.sync_copy(acc_sp.at[pl.ds(r, BATCH)], g0)
        pltpu.sync_copy(g0, out_hbm.at[c, pl.ds(r, BATCH)])


@jax.jit
def _agg_call(y, src, dst, ew):
    k = pl.kernel(
        _agg_body,
        out_type=jax.ShapeDtypeStruct((NC, NPAD, D), jnp.float32),
        mesh=plsc.VectorSubcoreMesh(**_MESH),
        scratch_types=(
            [pltpu.VMEM((BATCH,), jnp.int32) for _ in range(3)]
            + [pltpu.VMEM((BATCH,), jnp.int32) for _ in range(3)]
            + [pltpu.VMEM((BATCH,), jnp.float32) for _ in range(3)]
            + [pltpu.VMEM((BATCH, D), jnp.float32) for _ in range(3)]
            + [pltpu.VMEM_SHARED((NPAD, D), jnp.float32)]
            + [pltpu.SemaphoreType.DMA for _ in range(24)]
        ),
    )
    return k(y, src, dst, ew)


def _dinv_from(deg2):
    deg = deg2[0, :N] + deg2[1, :N] + 1.0
    safe = jnp.where(deg > 0.0, deg, 1.0)
    return jnp.where(deg > 0.0, lax.rsqrt(safe), 0.0)[:, None]


def _tc0_body(x_ref, w_ref, deg_ref, y_ref, dinv_ref):
    dinv = _dinv_from(deg_ref[...])
    xw = jnp.dot(x_ref[...], w_ref[...], preferred_element_type=jnp.float32)
    y_ref[...] = xw * dinv
    dinv_ref[...] = dinv


@jax.jit
def _tc0_call(x, W1, deg2):
    return pl.pallas_call(
        _tc0_body,
        out_shape=(
            jax.ShapeDtypeStruct((N, D), jnp.float32),
            jax.ShapeDtypeStruct((N, 1), jnp.float32),
        ),
    )(x, W1, deg2)


def _mix_body(agg_ref, y_ref, dinv_ref, b_ref, tp_ref, w_ref,
              temp_ref, ynext_ref):
    agg = agg_ref[0, :N, :] + agg_ref[1, :N, :]
    dinv = dinv_ref[...]
    out = dinv * (agg + y_ref[...]) + b_ref[...]
    temp = (1.0 - PRESERVE_C) * out + PRESERVE_C * tp_ref[...]
    temp_ref[...] = temp
    ynext_ref[...] = jnp.dot(temp, w_ref[...],
                             preferred_element_type=jnp.float32) * dinv


@jax.jit
def _mix_call(agg2, y, dinv, b, temp_prev, Wn):
    return pl.pallas_call(
        _mix_body,
        out_shape=(
            jax.ShapeDtypeStruct((N, D), jnp.float32),
            jax.ShapeDtypeStruct((N, D), jnp.float32),
        ),
    )(agg2, y, dinv, b, temp_prev, Wn)


def _fin_body(agg_ref, y_ref, dinv_ref, b_ref, tp_ref, temp_ref):
    agg = agg_ref[0, :N, :] + agg_ref[1, :N, :]
    out = dinv_ref[...] * (agg + y_ref[...]) + b_ref[...]
    temp_ref[...] = (1.0 - PRESERVE_C) * out + PRESERVE_C * tp_ref[...]


@jax.jit
def _fin_call(agg2, y, dinv, b, temp_prev):
    return pl.pallas_call(
        _fin_body,
        out_shape=jax.ShapeDtypeStruct((N, D), jnp.float32),
    )(agg2, y, dinv, b, temp_prev)


def kernel(skill_embed, adj_list, edge_attr, W1, b1, W2, b2, W3, b3):
    src = adj_list[0]
    dst = adj_list[1]
    dst3 = adj_list[1].reshape(NC * NS, NBATCH, BATCH)
    ew3 = edge_attr.reshape(NC * NS, NBATCH, BATCH)

    deg2 = _deg_call(dst3, ew3)
    y1, dinv = _tc0_call(skill_embed, W1, deg2)

    agg1 = _agg_call(y1, src, dst, edge_attr)
    temp1, y2 = _mix_call(agg1, y1, dinv, b1.reshape(1, D), skill_embed, W2)

    agg2 = _agg_call(y2, src, dst, edge_attr)
    temp2, y3 = _mix_call(agg2, y2, dinv, b2.reshape(1, D), temp1, W3)

    agg3 = _agg_call(y3, src, dst, edge_attr)
    return _fin_call(agg3, y3, dinv, b3.reshape(1, D), temp2)


# speedup vs baseline: 1.1177x; 1.0161x over previous
import functools

import jax
import jax.numpy as jnp
from jax import lax
from jax.experimental import pallas as pl
from jax.experimental.pallas import tpu as pltpu
from jax.experimental.pallas import tpu_sc as plsc

N = 10000
E = 320000
D = 128
NPAD = 10240
NC = 2
NS = 16
EDGES_PER_SC = E // NC
EDGES_PER_TILE = EDGES_PER_SC // NS
BATCH = 80
NBATCH = EDGES_PER_TILE // BATCH
ROWS_PER_TILE = NPAD // NS
FLUSH_ROWS = 128
PRESERVE_C = 0.1

_MESH = dict(core_axis_name="c", subcore_axis_name="s")


def _zero_vec_ref(ref, nwords):
    zeros = jnp.zeros((16,), jnp.float32)

    def body(i, _):
        ref[pl.ds(i * 16, 16)] = zeros
        return 0

    lax.fori_loop(0, nwords // 16, body, 0)


def _zero_mat_ref(ref, nrows, ncols):
    zeros = jnp.zeros((16,), jnp.float32)

    def body(r, _):
        row = ref.at[r]
        for v in range(ncols // 16):
            row[pl.ds(v * 16, 16)] = zeros
        return 0

    lax.fori_loop(0, nrows, body, 0)


def _deg_body(dst_hbm, ew_hbm, deg_out, idx_v, upd_v, stage_v, deg_sp, sem):
    c = lax.axis_index("c")
    s = lax.axis_index("s")
    row0 = s * ROWS_PER_TILE

    _zero_vec_ref(stage_v, ROWS_PER_TILE)
    pltpu.sync_copy(stage_v, deg_sp.at[pl.ds(row0, ROWS_PER_TILE)])

    t0 = c * NS + s
    pltpu.sync_copy(dst_hbm.at[t0], idx_v)
    pltpu.sync_copy(ew_hbm.at[t0], upd_v)
    plsc.subcore_barrier()

    def batch(j, _):
        pltpu.sync_copy(upd_v.at[j], deg_sp.at[idx_v.at[j]], add=True)
        return 0

    lax.fori_loop(0, NBATCH, batch, 0)
    plsc.subcore_barrier()

    pltpu.sync_copy(deg_sp.at[pl.ds(row0, ROWS_PER_TILE)], stage_v)
    pltpu.sync_copy(stage_v, deg_out.at[c, pl.ds(row0, ROWS_PER_TILE)])


@jax.jit
def _deg_call(dst2, ew2):
    k = pl.kernel(
        _deg_body,
        out_type=jax.ShapeDtypeStruct((NC, NPAD), jnp.float32),
        mesh=plsc.VectorSubcoreMesh(**_MESH),
        scratch_types=[
            pltpu.VMEM((NBATCH, BATCH), jnp.int32),
            pltpu.VMEM((NBATCH, BATCH), jnp.float32),
            pltpu.VMEM((ROWS_PER_TILE,), jnp.float32),
            pltpu.VMEM_SHARED((NPAD,), jnp.float32),
            pltpu.SemaphoreType.DMA,
        ],
    )
    return k(dst2, ew2)


_DNUMS = lax.GatherDimensionNumbers(
    offset_dims=(), collapsed_slice_dims=(0,), start_index_map=(0,))


def _agg_body(y_hbm, src_hbm, dst_hbm, ew_hbm, out_hbm,
              sb0, sb1, sb2, db0, db1, db2, eb0, eb1, eb2, g0, g1, g2,
              acc_sp,
              gs0, gs1, gs2, cs0, cs1, cs2, xs0, xs1, xs2,
              is0, is1, is2, es0, es1, es2, xs20, xs21, xs22,
              is20, is21, is22, es20, es21, es22):
    c = lax.axis_index("c")
    s = lax.axis_index("s")
    row0 = s * ROWS_PER_TILE
    t0 = c * NS + s
    ebase = t0 * EDGES_PER_TILE

    SB, XS = (sb0, sb1, sb2), (xs0, xs1, xs2)
    DB, IS = (db0, db1, db2), (is0, is1, is2)
    EB, ES = (eb0, eb1, eb2), (es0, es1, es2)
    G, GS = (g0, g1, g2), (gs0, gs1, gs2)
    CS = (cs0, cs1, cs2)
    XS2 = (xs20, xs21, xs22)
    IS2 = (is20, is21, is22)
    ES2 = (es20, es21, es22)

    _zero_mat_ref(g0, BATCH, D)
    for f in range(ROWS_PER_TILE // BATCH):
        pltpu.make_async_copy(
            g0, acc_sp.at[pl.ds(row0 + f * BATCH, BATCH)], gs0).start()
    for f in range(ROWS_PER_TILE // BATCH):
        pltpu.make_async_copy(
            g0, acc_sp.at[pl.ds(row0 + f * BATCH, BATCH)], gs0).wait()
    plsc.subcore_barrier()

    def edge_slice(hbm, j):
        return hbm.at[pl.ds(ebase + j * BATCH, BATCH)]

    def start_sidx(j, m):
        pltpu.make_async_copy(edge_slice(src_hbm, j), SB[m], XS[m]).start()

    def wait_sidx(j, m):
        pltpu.make_async_copy(edge_slice(src_hbm, j), SB[m], XS[m]).wait()

    def start_didx(j, m):
        pltpu.make_async_copy(edge_slice(dst_hbm, j), DB[m], IS[m]).start()

    def wait_didx(j, m):
        pltpu.make_async_copy(edge_slice(dst_hbm, j), DB[m], IS[m]).wait()

    def start_ew(j, m):
        pltpu.make_async_copy(edge_slice(ew_hbm, j), EB[m], ES[m]).start()

    def wait_ew(j, m):
        pltpu.make_async_copy(edge_slice(ew_hbm, j), EB[m], ES[m]).wait()

    GSPLIT = ((0, 40), (40, 40))
    GSEMS = (GS, XS2)

    def start_gather(m):
        for q, (o, n) in enumerate(GSPLIT):
            pltpu.make_async_copy(y_hbm.at[SB[m].at[pl.ds(o, n)]],
                                  G[m].at[pl.ds(o, n)], GSEMS[q][m]).start()

    def wait_gather(m):
        for q, (o, n) in enumerate(GSPLIT):
            pltpu.make_async_copy(y_hbm.at[SB[m].at[pl.ds(o, n)]],
                                  G[m].at[pl.ds(o, n)], GSEMS[q][m]).wait()

    def start_scatter(m):
        pltpu.make_async_copy(G[m], acc_sp.at[DB[m]], CS[m]).start(add=True)

    def wait_scatter(m):
        pltpu.make_async_copy(G[m], acc_sp.at[DB[m]], CS[m]).wait()

    def scale(j, k):
        wait_ew(j, k)
        gbuf, ebuf = G[k], EB[k]

        def group(g, _):
            ew16 = ebuf[pl.ds(g * 16, 16)]
            for l in range(16):
                bc = lax.gather(ew16, jnp.full((16, 1), l, jnp.int32),
                                _DNUMS, slice_sizes=(1,),
                                mode=lax.GatherScatterMode.PROMISE_IN_BOUNDS)
                gr = gbuf.at[g * 16 + l]
                for v in range(D // 16):
                    gr[pl.ds(v * 16, 16)] = gr[pl.ds(v * 16, 16)] * bc
            return 0

        lax.fori_loop(0, BATCH // 16, group, 0)

    def position(j, k, wait_sc=True, sidx2=True, nxt=True):
        k1, k2 = (k + 1) % 3, (k + 2) % 3
        if nxt:
            if wait_sc:
                wait_scatter(k1)
            wait_sidx(j + 1, k1)
            start_gather(k1)
        wait_gather(k)
        if sidx2:
            start_sidx(j + 2, k2)
        if nxt:
            start_didx(j + 1, k1)
            start_ew(j + 1, k1)
        scale(j, k)
        wait_didx(j, k)
        start_scatter(k)

    start_sidx(0, 0)
    start_sidx(1, 1)
    start_didx(0, 0)
    start_ew(0, 0)
    wait_sidx(0, 0)
    start_gather(0)

    position(0, 0, wait_sc=False)
    position(1, 1, wait_sc=False)

    def triple(t, _):
        j0 = 2 + 3 * t
        position(j0, 2)
        position(j0 + 1, 0)
        position(j0 + 2, 1)
        return 0

    lax.fori_loop(0, (NBATCH - 5) // 3, triple, 0)

    position(NBATCH - 3, 2)
    position(NBATCH - 2, 0, sidx2=False)
    position(NBATCH - 1, 1, sidx2=False, nxt=False)
    wait_scatter(2)
    wait_scatter(0)
    wait_scatter(1)

    plsc.subcore_barrier()

    NF = ROWS_PER_TILE // BATCH
    FB = (g0, g1)

    def fread(f, buf, sem):
        return pltpu.make_async_copy(
            acc_sp.at[pl.ds(row0 + f * BATCH, BATCH)], buf, sem)

    fread(0, g0, gs0).start()
    for f in range(NF):
        buf, sem = FB[f % 2], (gs0, gs1)[f % 2]
        fread(f, buf, sem).wait()
        if f + 1 < NF:
            nbuf, nsem = FB[(f + 1) % 2], (gs0, gs1)[(f + 1) % 2]
            fread(f + 1, nbuf, nsem).start()
        pltpu.sync_copy(buf, out_hbm.at[c, pl.ds(row0 + f * BATCH, BATCH)])


@jax.jit
def _agg_call(y, src, dst, ew):
    k = pl.kernel(
        _agg_body,
        out_type=jax.ShapeDtypeStruct((NC, NPAD, D), jnp.float32),
        mesh=plsc.VectorSubcoreMesh(**_MESH),
        scratch_types=(
            [pltpu.VMEM((BATCH,), jnp.int32) for _ in range(3)]
            + [pltpu.VMEM((BATCH,), jnp.int32) for _ in range(3)]
            + [pltpu.VMEM((BATCH,), jnp.float32) for _ in range(3)]
            + [pltpu.VMEM((BATCH, D), jnp.float32) for _ in range(3)]
            + [pltpu.VMEM_SHARED((NPAD, D), jnp.float32)]
            + [pltpu.SemaphoreType.DMA for _ in range(24)]
        ),
    )
    return k(y, src, dst, ew)


def _dinv_from(deg2):
    deg = deg2[0, :N] + deg2[1, :N] + 1.0
    safe = jnp.where(deg > 0.0, deg, 1.0)
    return jnp.where(deg > 0.0, lax.rsqrt(safe), 0.0)[:, None]


def _tc0_body(x_ref, w_ref, deg_ref, y_ref, dinv_ref):
    dinv = _dinv_from(deg_ref[...])
    xw = jnp.dot(x_ref[...], w_ref[...], preferred_element_type=jnp.float32)
    y_ref[...] = xw * dinv
    dinv_ref[...] = dinv


@jax.jit
def _tc0_call(x, W1, deg2):
    return pl.pallas_call(
        _tc0_body,
        out_shape=(
            jax.ShapeDtypeStruct((N, D), jnp.float32),
            jax.ShapeDtypeStruct((N, 1), jnp.float32),
        ),
    )(x, W1, deg2)


def _mix_body(agg_ref, y_ref, dinv_ref, b_ref, tp_ref, w_ref,
              temp_ref, ynext_ref):
    agg = agg_ref[0, :N, :] + agg_ref[1, :N, :]
    dinv = dinv_ref[...]
    out = dinv * (agg + y_ref[...]) + b_ref[...]
    temp = (1.0 - PRESERVE_C) * out + PRESERVE_C * tp_ref[...]
    temp_ref[...] = temp
    ynext_ref[...] = jnp.dot(temp, w_ref[...],
                             preferred_element_type=jnp.float32) * dinv


@jax.jit
def _mix_call(agg2, y, dinv, b, temp_prev, Wn):
    return pl.pallas_call(
        _mix_body,
        out_shape=(
            jax.ShapeDtypeStruct((N, D), jnp.float32),
            jax.ShapeDtypeStruct((N, D), jnp.float32),
        ),
    )(agg2, y, dinv, b, temp_prev, Wn)


def _fin_body(agg_ref, y_ref, dinv_ref, b_ref, tp_ref, temp_ref):
    agg = agg_ref[0, :N, :] + agg_ref[1, :N, :]
    out = dinv_ref[...] * (agg + y_ref[...]) + b_ref[...]
    temp_ref[...] = (1.0 - PRESERVE_C) * out + PRESERVE_C * tp_ref[...]


@jax.jit
def _fin_call(agg2, y, dinv, b, temp_prev):
    return pl.pallas_call(
        _fin_body,
        out_shape=jax.ShapeDtypeStruct((N, D), jnp.float32),
    )(agg2, y, dinv, b, temp_prev)


def kernel(skill_embed, adj_list, edge_attr, W1, b1, W2, b2, W3, b3):
    src = adj_list[0]
    dst = adj_list[1]
    dst3 = adj_list[1].reshape(NC * NS, NBATCH, BATCH)
    ew3 = edge_attr.reshape(NC * NS, NBATCH, BATCH)

    deg2 = _deg_call(dst3, ew3)
    y1, dinv = _tc0_call(skill_embed, W1, deg2)

    agg1 = _agg_call(y1, src, dst, edge_attr)
    temp1, y2 = _mix_call(agg1, y1, dinv, b1.reshape(1, D), skill_embed, W2)

    agg2 = _agg_call(y2, src, dst, edge_attr)
    temp2, y3 = _mix_call(agg2, y2, dinv, b2.reshape(1, D), temp1, W3)

    agg3 = _agg_call(y3, src, dst, edge_attr)
    return _fin_call(agg3, y3, dinv, b3.reshape(1, D), temp2)
